# baseline (speedup 1.0000x reference)
import jax
from jax.experimental import pallas as pl

_BLOCK_COLS = 4096


def _copy_body(x_ref, o_ref):
    o_ref[...] = x_ref[...]


def kernel(x):
    rows, cols = x.shape
    xt = x.T
    out = pl.pallas_call(
        _copy_body,
        grid=(rows // _BLOCK_COLS,),
        in_specs=[pl.BlockSpec((cols, _BLOCK_COLS), lambda i: (0, i))],
        out_specs=pl.BlockSpec((cols, _BLOCK_COLS), lambda i: (0, i)),
        out_shape=jax.ShapeDtypeStruct((cols, rows), x.dtype),
    )(xt)
    return out.T


# speedup vs baseline: 1.1277x; 1.1277x over previous
import jax
from jax.experimental import pallas as pl

_BLOCK_COLS = 16384


def _copy_body(x_ref, o_ref):
    o_ref[...] = x_ref[...]


def kernel(x):
    rows, cols = x.shape
    xt = x.T
    out = pl.pallas_call(
        _copy_body,
        grid=(rows // _BLOCK_COLS,),
        in_specs=[pl.BlockSpec((cols, _BLOCK_COLS), lambda i: (0, i))],
        out_specs=pl.BlockSpec((cols, _BLOCK_COLS), lambda i: (0, i)),
        out_shape=jax.ShapeDtypeStruct((cols, rows), x.dtype),
    )(xt)
    return out.T


# speedup vs baseline: 1.4320x; 1.2698x over previous
import jax
from jax.experimental import pallas as pl
from jax.experimental.pallas import tpu as pltpu

_N_CHUNKS = 8


def _copy_body(x_ref, o_ref, buf, in_sems, out_sems):
    chunk = x_ref.shape[0] // _N_CHUNKS
    for i in range(_N_CHUNKS):
        sl = pl.ds(i * chunk, chunk)
        pltpu.make_async_copy(x_ref.at[sl, :], buf.at[sl, :], in_sems.at[i]).start()
    for i in range(_N_CHUNKS):
        sl = pl.ds(i * chunk, chunk)
        pltpu.make_async_copy(x_ref.at[sl, :], buf.at[sl, :], in_sems.at[i]).wait()
        pltpu.make_async_copy(buf.at[sl, :], o_ref.at[sl, :], out_sems.at[i]).start()
    for i in range(_N_CHUNKS):
        sl = pl.ds(i * chunk, chunk)
        pltpu.make_async_copy(buf.at[sl, :], o_ref.at[sl, :], out_sems.at[i]).wait()


def kernel(x):
    rows, cols = x.shape
    xt = x.T
    out = pl.pallas_call(
        _copy_body,
        in_specs=[pl.BlockSpec(memory_space=pl.ANY)],
        out_specs=pl.BlockSpec(memory_space=pl.ANY),
        out_shape=jax.ShapeDtypeStruct((cols, rows), x.dtype),
        scratch_shapes=[
            pltpu.VMEM((cols, rows), x.dtype),
            pltpu.SemaphoreType.DMA((_N_CHUNKS,)),
            pltpu.SemaphoreType.DMA((_N_CHUNKS,)),
        ],
    )(xt)
    return out.T
